# baseline (speedup 1.0000x reference)
import functools
import jax
import jax.numpy as jnp
from jax import lax
from jax.experimental import pallas as pl
from jax.experimental.pallas import tpu as pltpu
from jax.experimental.pallas import tpu_sc as plsc

T_HIST = 12; T_FORE = 6; T_ALL = 18
M = 8; STATIC = 8; I2D = 16; NWPD = 32; HID = 64
N = 10000; ES = 160000; EX = 40000

NPAD = 10240
NW = 32
C = 128
NCH_S = 40
NCH_X = 10
ESP = NW * NCH_S * C
EXP = NW * NCH_X * C
ROWS_PER_SUB = NPAD // 16


def _make_spmm(W, nch):
    mesh = plsc.VectorSubcoreMesh(core_axis_name="c", subcore_axis_name="s")

    @functools.partial(
        pl.kernel, mesh=mesh,
        out_type=jax.ShapeDtypeStruct((2, NPAD, W), jnp.float32),
        scratch_types=[
            pltpu.VMEM((nch, C), jnp.int32),
            pltpu.VMEM((nch, C), jnp.int32),
            pltpu.VMEM((C * 16,), jnp.float32),
            pltpu.VMEM((C, W), jnp.float32),
            pltpu.VMEM_SHARED((NPAD, W), jnp.float32),
            pltpu.SemaphoreType.DMA,
        ],
    )
    def k(cur_hbm, src_hbm, dst_hbm, w_hbm, zeros_hbm, out_hbm,
          src_v, dst_v, w_v, rows_v, acc_s, sem):
        cid = lax.axis_index("c")
        sid = lax.axis_index("s")
        wid = sid * 2 + cid
        zb = sid * ROWS_PER_SUB
        pltpu.sync_copy(zeros_hbm.at[pl.ds(zb, ROWS_PER_SUB)],
                        acc_s.at[pl.ds(zb, ROWS_PER_SUB)])
        pltpu.sync_copy(src_hbm.at[wid], src_v)
        pltpu.sync_copy(dst_hbm.at[wid], dst_v)
        plsc.subcore_barrier()

        def chunk(j, carry):
            cp = pltpu.async_copy(cur_hbm.at[src_v.at[j]], rows_v, sem)
            pltpu.sync_copy(w_hbm.at[wid, j], w_v)
            cp.wait()

            def row(i, c2):
                wv = w_v[pl.ds(i * 16, 16)]
                for col in range(W // 16):
                    sl = pl.ds(col * 16, 16)
                    rows_v[i, sl] = rows_v[i, sl] * wv
                return c2

            lax.fori_loop(0, C, row, 0, unroll=8)
            pltpu.sync_copy(rows_v, acc_s.at[dst_v.at[j]], add=True)
            return carry

        lax.fori_loop(0, nch, chunk, 0)
        plsc.subcore_barrier()
        pltpu.sync_copy(acc_s.at[pl.ds(zb, ROWS_PER_SUB)],
                        out_hbm.at[cid, pl.ds(zb, ROWS_PER_SUB)])

    return k


_spmm_cache = {}


def _spmm(cur, src3, dst3, w16, zeros):
    nch = src3.shape[1]
    key = (cur.shape[1], nch)
    if key not in _spmm_cache:
        _spmm_cache[key] = _make_spmm(cur.shape[1], nch)
    return _spmm_cache[key](cur, src3, dst3,
                            w16.reshape(NW, nch, C * 16), zeros)


def _mm_body(a_ref, w_ref, b_ref, o_ref, *, act):
    o = jnp.dot(a_ref[...], w_ref[...],
                preferred_element_type=jnp.float32) + b_ref[...]
    if act == "tanh":
        o = jnp.tanh(o)
    o_ref[...] = o


def _matmul(a, w, b, act=None, br=400):
    R, D = a.shape
    F = w.shape[1]
    return pl.pallas_call(
        functools.partial(_mm_body, act=act),
        grid=(R // br,),
        in_specs=[
            pl.BlockSpec((br, D), lambda i: (i, 0)),
            pl.BlockSpec((D, F), lambda i: (0, 0)),
            pl.BlockSpec((1, F), lambda i: (0, 0)),
        ],
        out_specs=pl.BlockSpec((br, F), lambda i: (i, 0)),
        out_shape=jax.ShapeDtypeStruct((R, F), jnp.float32),
    )(a, w, b.reshape(1, F))


def _ew_call(body, ins, n_out, F, br=400):
    R = ins[0].shape[0]
    specs = [pl.BlockSpec((br, x.shape[1]), lambda i: (i, 0)) for x in ins]
    shapes = [jax.ShapeDtypeStruct((R, F), jnp.float32)] * n_out
    out_specs = [pl.BlockSpec((br, F), lambda i: (i, 0))] * n_out
    if n_out == 1:
        shapes, out_specs = shapes[0], out_specs[0]
    return pl.pallas_call(
        body, grid=(R // br,), in_specs=specs,
        out_specs=out_specs, out_shape=shapes,
    )(*ins)


def _inv3_body(g_ref, s0_ref, s1_ref, o_ref):
    o_ref[...] = 1.0 / (g_ref[...] + s0_ref[...] + s1_ref[...])


def _combine_body(g_ref, s0_ref, s1_ref, inv_ref, o_ref):
    o_ref[...] = g_ref[...] + (s0_ref[...] + s1_ref[...]) * inv_ref[...]


def _nwp_body(s0_ref, s1_ref, inv_ref, o_ref):
    o_ref[...] = (s0_ref[...] + s1_ref[...]) * inv_ref[...]


def _ru_body(g_ref, s0_ref, s1_ref, inv_ref, h_ref, rh_ref, u_ref):
    ru = jax.nn.sigmoid(
        g_ref[...] + (s0_ref[...] + s1_ref[...]) * inv_ref[...])
    rh_ref[...] = ru[:, :HID] * h_ref[...]
    u_ref[...] = ru[:, HID:]


def _h_body(g_ref, s0_ref, s1_ref, inv_ref, u_ref, h_ref, o_ref):
    c = jnp.tanh(g_ref[...] + (s0_ref[...] + s1_ref[...]) * inv_ref[...])
    u = u_ref[...]
    o_ref[...] = u * h_ref[...] + (1.0 - u) * c


def _wsig_body(ea_ref, we_ref, o_ref, *, n_valid, br):
    pid = pl.program_id(0)
    w = jax.nn.sigmoid(
        jnp.sum(ea_ref[...] * we_ref[...], axis=-1, keepdims=True))
    rid = pid * br + lax.broadcasted_iota(jnp.int32, (br, 1), 0)
    o_ref[...] = jnp.broadcast_to(jnp.where(rid < n_valid, w, 0.0), (br, 16))


def _expneg_body(a_ref, o_ref, *, n_valid, br):
    pid = pl.program_id(0)
    rid = pid * br + lax.broadcasted_iota(jnp.int32, (br, 1), 0)
    o_ref[...] = jnp.broadcast_to(
        jnp.where(rid < n_valid, jnp.exp(-a_ref[...]), 0.0), (br, 16))


def _mask_body(y_ref, m_ref, o_ref):
    o_ref[...] = y_ref[...] * m_ref[...]


def _pad_edges(src, dst, epad):
    e = src.shape[0]
    src = jnp.pad(src.astype(jnp.int32), (0, epad - e))
    dst = jnp.pad(dst.astype(jnp.int32), (0, epad - e))
    return src.reshape(NW, -1, C), dst.reshape(NW, -1, C)


def _gru(x, h, wru, bru, wc, bc, s3, d3, wn16, inv, z128):
    z = jnp.concatenate([x, h], axis=-1)
    g = _matmul(z, wru, bru)
    g0, g1, g2 = g[:, :128], g[:, 128:256], g[:, 256:384]
    p = _spmm(g2, s3, d3, wn16, z128)
    p1 = _ew_call(_combine_body, [g1, p[0, :N], p[1, :N], inv], 1, 128)
    p = _spmm(p1, s3, d3, wn16, z128)
    rh, u = _ew_call(_ru_body, [g0, p[0, :N], p[1, :N], inv, h], 2, HID)
    z2 = jnp.concatenate([x, rh], axis=-1)
    gc = _matmul(z2, wc, bc)
    g0c, g1c, g2c = gc[:, :64], gc[:, 64:128], gc[:, 128:192]
    pc = _spmm(jnp.pad(g2c, ((0, 0), (0, 64))), s3, d3, wn16, z128)
    p1c = _ew_call(_combine_body,
                   [g1c, pc[0, :N, :64], pc[1, :N, :64], inv], 1, 64)
    pc = _spmm(jnp.pad(p1c, ((0, 0), (0, 64))), s3, d3, wn16, z128)
    return _ew_call(_h_body,
                    [g0c, pc[0, :N, :64], pc[1, :N, :64], inv, u, h], 1, HID)


def _wcat(wru, bru, wc, bc, d):
    wru3 = jnp.concatenate([wru[:d], wru[d:2 * d], wru[2 * d:]], axis=1)
    bru3 = jnp.concatenate([bru, jnp.zeros((256,), jnp.float32)])
    wc3 = jnp.concatenate([wc[:d], wc[d:2 * d], wc[2 * d:]], axis=1)
    bc3 = jnp.concatenate([bc, jnp.zeros((128,), jnp.float32)])
    return wru3, bru3, wc3, bc3


def kernel(x_station, static, icond2_x, ecmwf_x, s2s_edge_attr, i2s_edge_attr,
           e2s_edge_attr, We_i, We_e, Wn_i, bn_i, Wn_e, bn_e,
           enc0_Wru, enc0_bru, enc0_Wc, enc0_bc,
           enc1_Wru, enc1_bru, enc1_Wc, enc1_bc,
           dec0_Wru, dec0_bru, dec0_Wc, dec0_bc,
           dec1_Wru, dec1_bru, dec1_Wc, dec1_bc,
           W_out, b_out, s2s_edge_index, i2s_edge_index, e2s_edge_index,
           target_mask):
    f32 = jnp.float32
    z128 = jnp.zeros((NPAD, 128), f32)
    ones128 = jnp.ones((N, 128), f32)
    eps128 = jnp.full((NPAD, 128), 1e-6, f32)

    ss3, sd3 = _pad_edges(s2s_edge_index[0], s2s_edge_index[1], ESP)
    is3, id3 = _pad_edges(i2s_edge_index[0], i2s_edge_index[1], EXP)
    es3, ed3 = _pad_edges(e2s_edge_index[0], e2s_edge_index[1], EXP)

    a0 = jnp.pad(s2s_edge_attr[:, 0], (0, ESP - ES)).reshape(ESP, 1)
    ew16 = _ew_call(functools.partial(_expneg_body, n_valid=ES, br=2048),
                    [a0], 1, 16, br=2048)
    eai = jnp.pad(i2s_edge_attr, ((0, EXP - EX), (0, 0)))
    eae = jnp.pad(e2s_edge_attr, ((0, EXP - EX), (0, 0)))
    wi16 = _ew_call(functools.partial(_wsig_body, n_valid=EX, br=2048),
                    [eai, jnp.broadcast_to(We_i.reshape(1, 4), (EXP, 4))],
                    1, 16, br=2048)
    we16 = _ew_call(functools.partial(_wsig_body, n_valid=EX, br=2048),
                    [eae, jnp.broadcast_to(We_e.reshape(1, 4), (EXP, 4))],
                    1, 16, br=2048)

    degp = _spmm(ones128, ss3, sd3, ew16, z128)
    invdeg = _ew_call(_inv3_body, [eps128, degp[0], degp[1]], 1, 128,
                      br=512)[:N, :1]
    denip = _spmm(ones128, is3, id3, wi16, z128)
    invdeni = _ew_call(_inv3_body, [eps128, denip[0], denip[1]], 1, 128,
                       br=512)[:N, :1]
    denep = _spmm(ones128, es3, ed3, we16, z128)
    invdene = _ew_call(_inv3_body, [eps128, denep[0], denep[1]], 1, 128,
                       br=512)[:N, :1]

    def nwp_all(feat, s3, d3, w16, invden, Wn, bn):
        parts = []
        for lo, tg in ((0, 8), (8, 8), (16, 2)):
            fa = feat[:, lo:lo + tg, :].reshape(N, tg * I2D)
            fa = jnp.pad(fa, ((0, 0), (0, 128 - tg * I2D)))
            p = _spmm(fa, s3, d3, w16, z128)
            part = _ew_call(_nwp_body,
                            [p[0, :N, :tg * I2D], p[1, :N, :tg * I2D],
                             invden], 1, tg * I2D)
            parts.append(part.reshape(N, tg, I2D))
        num = jnp.concatenate(parts, axis=1)
        out = _matmul(num.reshape(N * T_ALL, I2D), Wn, bn, act="tanh")
        return out.reshape(N, T_ALL, NWPD)

    nwp_i = nwp_all(icond2_x, is3, id3, wi16, invdeni, Wn_i, bn_i)
    nwp_e = nwp_all(ecmwf_x, es3, ed3, we16, invdene, Wn_e, bn_e)

    e0 = _wcat(enc0_Wru, enc0_bru, enc0_Wc, enc0_bc, M + 2 * NWPD + STATIC + HID)
    e1 = _wcat(enc1_Wru, enc1_bru, enc1_Wc, enc1_bc, 2 * HID)
    d0 = _wcat(dec0_Wru, dec0_bru, dec0_Wc, dec0_bc, 1 + 2 * NWPD + STATIC + HID)
    d1 = _wcat(dec1_Wru, dec1_bru, dec1_Wc, dec1_bc, 2 * HID)

    h0 = jnp.zeros((N, HID), f32)
    h1 = jnp.zeros((N, HID), f32)
    for t in range(T_HIST):
        inp = jnp.concatenate(
            [x_station[:, t, :], nwp_i[:, t], nwp_e[:, t], static], axis=-1)
        h0 = _gru(inp, h0, *e0, ss3, sd3, ew16, invdeg, z128)
        h1 = _gru(h0, h1, *e1, ss3, sd3, ew16, invdeg, z128)
    y = x_station[:, -1, 0:1]
    preds = []
    for t in range(T_FORE):
        ti = T_HIST + t
        inp = jnp.concatenate([y, nwp_i[:, ti], nwp_e[:, ti], static], axis=-1)
        h0 = _gru(inp, h0, *d0, ss3, sd3, ew16, invdeg, z128)
        h1 = _gru(h0, h1, *d1, ss3, sd3, ew16, invdeg, z128)
        y = _matmul(h1, W_out, b_out)
        preds.append(y)
    ycat = jnp.concatenate(preds, axis=1)
    maskf = target_mask.astype(f32).reshape(N, 1)
    return _ew_call(_mask_body, [ycat, maskf], 1, T_FORE)


# speedup vs baseline: 1.0940x; 1.0940x over previous
import functools
import jax
import jax.numpy as jnp
from jax import lax
from jax.experimental import pallas as pl
from jax.experimental.pallas import tpu as pltpu
from jax.experimental.pallas import tpu_sc as plsc

T_HIST = 12; T_FORE = 6; T_ALL = 18
M = 8; STATIC = 8; I2D = 16; NWPD = 32; HID = 64
N = 10000; ES = 160000; EX = 40000

NPAD = 10240
NW = 32
C = 128
NCH_S = 40
NCH_X = 10
ESP = NW * NCH_S * C
EXP = NW * NCH_X * C
ROWS_PER_SUB = NPAD // 16


def _make_spmm(W, nch):
    mesh = plsc.VectorSubcoreMesh(core_axis_name="c", subcore_axis_name="s")

    @functools.partial(
        pl.kernel, mesh=mesh,
        out_type=jax.ShapeDtypeStruct((2, NPAD, W), jnp.float32),
        scratch_types=[
            pltpu.VMEM((nch, C), jnp.int32),
            pltpu.VMEM((nch, C), jnp.int32),
            pltpu.VMEM((2, C * 16), jnp.float32),
            pltpu.VMEM((C, W), jnp.float32),
            pltpu.VMEM((C, W), jnp.float32),
            pltpu.VMEM_SHARED((NPAD, W), jnp.float32),
            pltpu.SemaphoreType.DMA,
            pltpu.SemaphoreType.DMA,
            pltpu.SemaphoreType.DMA,
        ],
    )
    def k(cur_hbm, src_hbm, dst_hbm, w_hbm, zeros_hbm, out_hbm,
          src_v, dst_v, w_v, rows0_v, rows1_v, acc_s, sem0, sem1, ssem):
        cid = lax.axis_index("c")
        sid = lax.axis_index("s")
        wid = sid * 2 + cid
        zb = sid * ROWS_PER_SUB
        pltpu.sync_copy(zeros_hbm.at[pl.ds(zb, ROWS_PER_SUB)],
                        acc_s.at[pl.ds(zb, ROWS_PER_SUB)])
        pltpu.sync_copy(src_hbm.at[wid], src_v)
        pltpu.sync_copy(dst_hbm.at[wid], dst_v)
        plsc.subcore_barrier()

        def scale(rows_v, wrow):
            def row(i, c2):
                wv = w_v[wrow, pl.ds(i * 16, 16)]
                for col in range(W // 16):
                    sl = pl.ds(col * 16, 16)
                    rows_v[i, sl] = rows_v[i, sl] * wv
                return c2

            lax.fori_loop(0, C, row, 0)

        def chunk(jj, carry):
            j0 = jj * 2
            g0 = pltpu.async_copy(cur_hbm.at[src_v.at[j0]], rows0_v, sem0)
            g1 = pltpu.async_copy(cur_hbm.at[src_v.at[j0 + 1]], rows1_v, sem1)
            pltpu.sync_copy(w_hbm.at[wid, pl.ds(j0, 2)], w_v)
            g0.wait()
            scale(rows0_v, 0)
            s0 = pltpu.async_copy(rows0_v, acc_s.at[dst_v.at[j0]], ssem,
                                  add=True)
            g1.wait()
            scale(rows1_v, 1)
            s0.wait()
            pltpu.sync_copy(rows1_v, acc_s.at[dst_v.at[j0 + 1]], add=True)
            return carry

        lax.fori_loop(0, nch // 2, chunk, 0)
        plsc.subcore_barrier()
        pltpu.sync_copy(acc_s.at[pl.ds(zb, ROWS_PER_SUB)],
                        out_hbm.at[cid, pl.ds(zb, ROWS_PER_SUB)])

    return k


_spmm_cache = {}


def _spmm(cur, src3, dst3, w16, zeros):
    nch = src3.shape[1]
    key = (cur.shape[1], nch)
    if key not in _spmm_cache:
        _spmm_cache[key] = _make_spmm(cur.shape[1], nch)
    return _spmm_cache[key](cur, src3, dst3,
                            w16.reshape(NW, nch, C * 16), zeros)


def _mm_body(a_ref, w_ref, b_ref, o_ref, *, act):
    o = jnp.dot(a_ref[...], w_ref[...],
                preferred_element_type=jnp.float32) + b_ref[...]
    if act == "tanh":
        o = jnp.tanh(o)
    o_ref[...] = o


def _matmul(a, w, b, act=None, br=400):
    R, D = a.shape
    F = w.shape[1]
    return pl.pallas_call(
        functools.partial(_mm_body, act=act),
        grid=(R // br,),
        in_specs=[
            pl.BlockSpec((br, D), lambda i: (i, 0)),
            pl.BlockSpec((D, F), lambda i: (0, 0)),
            pl.BlockSpec((1, F), lambda i: (0, 0)),
        ],
        out_specs=pl.BlockSpec((br, F), lambda i: (i, 0)),
        out_shape=jax.ShapeDtypeStruct((R, F), jnp.float32),
    )(a, w, b.reshape(1, F))


def _ew_call(body, ins, n_out, F, br=400):
    R = ins[0].shape[0]
    specs = [pl.BlockSpec((br, x.shape[1]), lambda i: (i, 0)) for x in ins]
    shapes = [jax.ShapeDtypeStruct((R, F), jnp.float32)] * n_out
    out_specs = [pl.BlockSpec((br, F), lambda i: (i, 0))] * n_out
    if n_out == 1:
        shapes, out_specs = shapes[0], out_specs[0]
    return pl.pallas_call(
        body, grid=(R // br,), in_specs=specs,
        out_specs=out_specs, out_shape=shapes,
    )(*ins)


def _inv3_body(g_ref, s0_ref, s1_ref, o_ref):
    o_ref[...] = 1.0 / (g_ref[...] + s0_ref[...] + s1_ref[...])


def _combine_body(g_ref, s0_ref, s1_ref, inv_ref, o_ref):
    o_ref[...] = g_ref[...] + (s0_ref[...] + s1_ref[...]) * inv_ref[...]


def _nwp_body(s0_ref, s1_ref, inv_ref, o_ref):
    o_ref[...] = (s0_ref[...] + s1_ref[...]) * inv_ref[...]


def _ru_body(g_ref, s0_ref, s1_ref, inv_ref, h_ref, rh_ref, u_ref):
    ru = jax.nn.sigmoid(
        g_ref[...] + (s0_ref[...] + s1_ref[...]) * inv_ref[...])
    rh_ref[...] = ru[:, :HID] * h_ref[...]
    u_ref[...] = ru[:, HID:]


def _h_body(g_ref, s0_ref, s1_ref, inv_ref, u_ref, h_ref, o_ref):
    c = jnp.tanh(g_ref[...] + (s0_ref[...] + s1_ref[...]) * inv_ref[...])
    u = u_ref[...]
    o_ref[...] = u * h_ref[...] + (1.0 - u) * c


def _wsig_body(ea_ref, we_ref, o_ref, *, n_valid, br):
    pid = pl.program_id(0)
    w = jax.nn.sigmoid(
        jnp.sum(ea_ref[...] * we_ref[...], axis=-1, keepdims=True))
    rid = pid * br + lax.broadcasted_iota(jnp.int32, (br, 1), 0)
    o_ref[...] = jnp.broadcast_to(jnp.where(rid < n_valid, w, 0.0), (br, 16))


def _expneg_body(a_ref, o_ref, *, n_valid, br):
    pid = pl.program_id(0)
    rid = pid * br + lax.broadcasted_iota(jnp.int32, (br, 1), 0)
    o_ref[...] = jnp.broadcast_to(
        jnp.where(rid < n_valid, jnp.exp(-a_ref[...]), 0.0), (br, 16))


def _mask_body(y_ref, m_ref, o_ref):
    o_ref[...] = y_ref[...] * m_ref[...]


def _pad_edges(src, dst, epad):
    e = src.shape[0]
    src = jnp.pad(src.astype(jnp.int32), (0, epad - e))
    dst = jnp.pad(dst.astype(jnp.int32), (0, epad - e))
    return src.reshape(NW, -1, C), dst.reshape(NW, -1, C)


def _gru(x, h, wru, bru, wc, bc, s3, d3, wn16, inv, z128):
    z = jnp.concatenate([x, h], axis=-1)
    g = _matmul(z, wru, bru)
    g0, g1, g2 = g[:, :128], g[:, 128:256], g[:, 256:384]
    p = _spmm(g2, s3, d3, wn16, z128)
    p1 = _ew_call(_combine_body, [g1, p[0, :N], p[1, :N], inv], 1, 128)
    p = _spmm(p1, s3, d3, wn16, z128)
    rh, u = _ew_call(_ru_body, [g0, p[0, :N], p[1, :N], inv, h], 2, HID)
    z2 = jnp.concatenate([x, rh], axis=-1)
    gc = _matmul(z2, wc, bc)
    g0c, g1c, g2c = gc[:, :64], gc[:, 64:128], gc[:, 128:192]
    pc = _spmm(jnp.pad(g2c, ((0, 0), (0, 64))), s3, d3, wn16, z128)
    p1c = _ew_call(_combine_body,
                   [g1c, pc[0, :N, :64], pc[1, :N, :64], inv], 1, 64)
    pc = _spmm(jnp.pad(p1c, ((0, 0), (0, 64))), s3, d3, wn16, z128)
    return _ew_call(_h_body,
                    [g0c, pc[0, :N, :64], pc[1, :N, :64], inv, u, h], 1, HID)


def _wcat(wru, bru, wc, bc, d):
    wru3 = jnp.concatenate([wru[:d], wru[d:2 * d], wru[2 * d:]], axis=1)
    bru3 = jnp.concatenate([bru, jnp.zeros((256,), jnp.float32)])
    wc3 = jnp.concatenate([wc[:d], wc[d:2 * d], wc[2 * d:]], axis=1)
    bc3 = jnp.concatenate([bc, jnp.zeros((128,), jnp.float32)])
    return wru3, bru3, wc3, bc3


def kernel(x_station, static, icond2_x, ecmwf_x, s2s_edge_attr, i2s_edge_attr,
           e2s_edge_attr, We_i, We_e, Wn_i, bn_i, Wn_e, bn_e,
           enc0_Wru, enc0_bru, enc0_Wc, enc0_bc,
           enc1_Wru, enc1_bru, enc1_Wc, enc1_bc,
           dec0_Wru, dec0_bru, dec0_Wc, dec0_bc,
           dec1_Wru, dec1_bru, dec1_Wc, dec1_bc,
           W_out, b_out, s2s_edge_index, i2s_edge_index, e2s_edge_index,
           target_mask):
    f32 = jnp.float32
    z128 = jnp.zeros((NPAD, 128), f32)
    ones128 = jnp.ones((N, 128), f32)
    eps128 = jnp.full((NPAD, 128), 1e-6, f32)

    ss3, sd3 = _pad_edges(s2s_edge_index[0], s2s_edge_index[1], ESP)
    is3, id3 = _pad_edges(i2s_edge_index[0], i2s_edge_index[1], EXP)
    es3, ed3 = _pad_edges(e2s_edge_index[0], e2s_edge_index[1], EXP)

    a0 = jnp.pad(s2s_edge_attr[:, 0], (0, ESP - ES)).reshape(ESP, 1)
    ew16 = _ew_call(functools.partial(_expneg_body, n_valid=ES, br=2048),
                    [a0], 1, 16, br=2048)
    eai = jnp.pad(i2s_edge_attr, ((0, EXP - EX), (0, 0)))
    eae = jnp.pad(e2s_edge_attr, ((0, EXP - EX), (0, 0)))
    wi16 = _ew_call(functools.partial(_wsig_body, n_valid=EX, br=2048),
                    [eai, jnp.broadcast_to(We_i.reshape(1, 4), (EXP, 4))],
                    1, 16, br=2048)
    we16 = _ew_call(functools.partial(_wsig_body, n_valid=EX, br=2048),
                    [eae, jnp.broadcast_to(We_e.reshape(1, 4), (EXP, 4))],
                    1, 16, br=2048)

    degp = _spmm(ones128, ss3, sd3, ew16, z128)
    invdeg = _ew_call(_inv3_body, [eps128, degp[0], degp[1]], 1, 128,
                      br=512)[:N, :1]
    denip = _spmm(ones128, is3, id3, wi16, z128)
    invdeni = _ew_call(_inv3_body, [eps128, denip[0], denip[1]], 1, 128,
                       br=512)[:N, :1]
    denep = _spmm(ones128, es3, ed3, we16, z128)
    invdene = _ew_call(_inv3_body, [eps128, denep[0], denep[1]], 1, 128,
                       br=512)[:N, :1]

    def nwp_all(feat, s3, d3, w16, invden, Wn, bn):
        parts = []
        for lo, tg in ((0, 8), (8, 8), (16, 2)):
            fa = feat[:, lo:lo + tg, :].reshape(N, tg * I2D)
            fa = jnp.pad(fa, ((0, 0), (0, 128 - tg * I2D)))
            p = _spmm(fa, s3, d3, w16, z128)
            part = _ew_call(_nwp_body,
                            [p[0, :N, :tg * I2D], p[1, :N, :tg * I2D],
                             invden], 1, tg * I2D)
            parts.append(part.reshape(N, tg, I2D))
        num = jnp.concatenate(parts, axis=1)
        out = _matmul(num.reshape(N * T_ALL, I2D), Wn, bn, act="tanh")
        return out.reshape(N, T_ALL, NWPD)

    nwp_i = nwp_all(icond2_x, is3, id3, wi16, invdeni, Wn_i, bn_i)
    nwp_e = nwp_all(ecmwf_x, es3, ed3, we16, invdene, Wn_e, bn_e)

    e0 = _wcat(enc0_Wru, enc0_bru, enc0_Wc, enc0_bc, M + 2 * NWPD + STATIC + HID)
    e1 = _wcat(enc1_Wru, enc1_bru, enc1_Wc, enc1_bc, 2 * HID)
    d0 = _wcat(dec0_Wru, dec0_bru, dec0_Wc, dec0_bc, 1 + 2 * NWPD + STATIC + HID)
    d1 = _wcat(dec1_Wru, dec1_bru, dec1_Wc, dec1_bc, 2 * HID)

    h0 = jnp.zeros((N, HID), f32)
    h1 = jnp.zeros((N, HID), f32)
    for t in range(T_HIST):
        inp = jnp.concatenate(
            [x_station[:, t, :], nwp_i[:, t], nwp_e[:, t], static], axis=-1)
        h0 = _gru(inp, h0, *e0, ss3, sd3, ew16, invdeg, z128)
        h1 = _gru(h0, h1, *e1, ss3, sd3, ew16, invdeg, z128)
    y = x_station[:, -1, 0:1]
    preds = []
    for t in range(T_FORE):
        ti = T_HIST + t
        inp = jnp.concatenate([y, nwp_i[:, ti], nwp_e[:, ti], static], axis=-1)
        h0 = _gru(inp, h0, *d0, ss3, sd3, ew16, invdeg, z128)
        h1 = _gru(h0, h1, *d1, ss3, sd3, ew16, invdeg, z128)
        y = _matmul(h1, W_out, b_out)
        preds.append(y)
    ycat = jnp.concatenate(preds, axis=1)
    maskf = target_mask.astype(f32).reshape(N, 1)
    return _ew_call(_mask_body, [ycat, maskf], 1, T_FORE)
